# baseline (speedup 1.0000x reference)
import functools

import jax
import jax.numpy as jnp
from jax import lax
from jax.experimental import pallas as pl
from jax.experimental.pallas import tpu as pltpu
from jax.experimental.pallas import tpu_sc as plsc

ROWS = 4096
FIELD_NUM = 26
EMBED_DIM = 64
COLS = FIELD_NUM * EMBED_DIM
NCAND = 128
L = 16
NC = 2
NS = 16
NW = NC * NS
NGRP = COLS // 8


def _sc_body(x_hbm, cand_hbm, o_hbm, cand_v, zeros_v, sem_cp, sem_z):
    wid = lax.axis_index("s") * NC + lax.axis_index("c")
    pltpu.sync_copy(cand_hbm, cand_v)

    def zfill(t, c2):
        for kk in range(8):
            zeros_v[0, pl.ds(t * 128 + kk * L, L)] = jnp.zeros((L,), jnp.float32)
        return c2

    lax.fori_loop(0, ROWS // 128, zfill, 0)

    lo16 = wid < 16
    g0 = jnp.where(lo16, wid * 7, 112 + (wid - 16) * 6)
    ngrp = jnp.where(lo16, 7, 6)
    r0 = g0 * 8
    nrows = ngrp * 8

    @pl.when(lo16)
    def _():
        pltpu.async_copy(x_hbm.at[pl.ds(r0, 56)], o_hbm.at[pl.ds(r0, 56)], sem_cp)
        pltpu.make_async_copy(x_hbm.at[pl.ds(r0, 56)], o_hbm.at[pl.ds(r0, 56)], sem_cp).wait()

    @pl.when(jnp.logical_not(lo16))
    def _():
        pltpu.async_copy(x_hbm.at[pl.ds(r0, 48)], o_hbm.at[pl.ds(r0, 48)], sem_cp)
        pltpu.make_async_copy(x_hbm.at[pl.ds(r0, 48)], o_hbm.at[pl.ds(r0, 48)], sem_cp).wait()

    for j in range(NCAND // L):
        cv = cand_v[pl.ds(j * L, L)]
        inspan = jnp.where(jnp.logical_and(cv >= r0, cv < r0 + nrows), 1, 0)
        for i in range(L):
            c = cv[i]

            @pl.when(inspan[i] == 1)
            def _(c=c):
                pltpu.async_copy(
                    zeros_v.at[pl.ds(0, 1)], o_hbm.at[pl.ds(c, 1)], sem_z
                )
                pltpu.make_async_copy(
                    zeros_v.at[pl.ds(0, 1)], o_hbm.at[pl.ds(c, 1)], sem_z
                ).wait()


def kernel(embed, cand):
    mesh = plsc.VectorSubcoreMesh(core_axis_name="c", subcore_axis_name="s")
    run = functools.partial(
        pl.kernel,
        mesh=mesh,
        out_type=jax.ShapeDtypeStruct((COLS, ROWS), jnp.float32),
        scratch_types=[
            pltpu.VMEM((NCAND,), jnp.int32),
            pltpu.VMEM((1, ROWS), jnp.float32),
            pltpu.SemaphoreType.DMA,
            pltpu.SemaphoreType.DMA,
        ],
        compiler_params=pltpu.CompilerParams(
            use_tc_tiling_on_sc=True, needs_layout_passes=False
        ),
    )(_sc_body)
    xt = embed.transpose(1, 2, 0).reshape(COLS, ROWS)
    out = run(xt, cand)
    return out.reshape(FIELD_NUM, EMBED_DIM, ROWS).transpose(2, 0, 1)


# speedup vs baseline: 19.9173x; 19.9173x over previous
import functools

import jax
import jax.numpy as jnp
from jax import lax
from jax.experimental import pallas as pl
from jax.experimental.pallas import tpu as pltpu
from jax.experimental.pallas import tpu_sc as plsc

ROWS = 4096
FIELD_NUM = 26
EMBED_DIM = 64
COLS = FIELD_NUM * EMBED_DIM
NCAND = 128
L = 16
NC = 2
NS = 16
NW = NC * NS
NGRP = COLS // 8
GPW = -(-NGRP // NW)
NBUF = 3


def _zero_hit_rows(buf, cand_v, g):
    hm = jnp.zeros((L,), jnp.int32)
    one = jnp.full((L,), 1, jnp.int32)
    for j in range(NCAND // L):
        cv = cand_v[pl.ds(j * L, L)]
        gv = lax.shift_right_logical(cv, 3)
        sv = lax.bitwise_and(cv, jnp.full((L,), 7, jnp.int32))
        hm = hm | jnp.where(gv == g, lax.shift_left(one, sv), 0)
    for s in range(8):
        cnt = plsc.all_reduce_population_count(
            lax.bitwise_and(lax.shift_right_logical(hm, s), 1) == 1
        )

        @pl.when(cnt[0] > 0)
        def _(s=s):
            def zrow(t, c2):
                for kk in range(8):
                    buf[s, pl.ds(t * 128 + kk * L, L)] = jnp.zeros((L,), jnp.float32)
                return c2

            lax.fori_loop(0, ROWS // 128, zrow, 0)


def _sc_body(x_hbm, cand_hbm, o_hbm, cand_v, b0, b1, b2, si0, si1, si2, so0, so1, so2):
    wid = lax.axis_index("s") * NC + lax.axis_index("c")
    pltpu.sync_copy(cand_hbm, cand_v)
    bufs = (b0, b1, b2)
    sins = (si0, si1, si2)
    souts = (so0, so1, so2)

    def g_of(k):
        return wid + NW * k

    def in_start(k):
        @pl.when(g_of(k) < NGRP)
        def _():
            pltpu.async_copy(x_hbm.at[pl.ds(g_of(k) * 8, 8)], bufs[k % NBUF], sins[k % NBUF])

    def in_wait(k):
        @pl.when(g_of(k) < NGRP)
        def _():
            pltpu.make_async_copy(
                x_hbm.at[pl.ds(g_of(k) * 8, 8)], bufs[k % NBUF], sins[k % NBUF]
            ).wait()

    def out_start(k):
        @pl.when(g_of(k) < NGRP)
        def _():
            pltpu.async_copy(bufs[k % NBUF], o_hbm.at[pl.ds(g_of(k) * 8, 8)], souts[k % NBUF])

    def out_wait(k):
        @pl.when(g_of(k) < NGRP)
        def _():
            pltpu.make_async_copy(
                bufs[k % NBUF], o_hbm.at[pl.ds(g_of(k) * 8, 8)], souts[k % NBUF]
            ).wait()

    in_start(0)
    in_start(1)
    for k in range(GPW):
        in_wait(k)
        if k + 2 < GPW:
            if k >= 1:
                out_wait(k - 1)
            in_start(k + 2)
        _zero_hit_rows(bufs[k % NBUF], cand_v, g_of(k))
        out_start(k)
    for k in range(max(0, GPW - 3), GPW):
        out_wait(k)


def kernel(embed, cand):
    mesh = plsc.VectorSubcoreMesh(core_axis_name="c", subcore_axis_name="s")
    run = functools.partial(
        pl.kernel,
        mesh=mesh,
        out_type=jax.ShapeDtypeStruct((COLS, ROWS), jnp.float32),
        scratch_types=[
            pltpu.VMEM((NCAND,), jnp.int32),
            pltpu.VMEM((8, ROWS), jnp.float32),
            pltpu.VMEM((8, ROWS), jnp.float32),
            pltpu.VMEM((8, ROWS), jnp.float32),
            pltpu.SemaphoreType.DMA,
            pltpu.SemaphoreType.DMA,
            pltpu.SemaphoreType.DMA,
            pltpu.SemaphoreType.DMA,
            pltpu.SemaphoreType.DMA,
            pltpu.SemaphoreType.DMA,
        ],
        compiler_params=pltpu.CompilerParams(
            use_tc_tiling_on_sc=True, needs_layout_passes=False
        ),
    )(_sc_body)
    xt = embed.transpose(1, 2, 0).reshape(COLS, ROWS)
    out = run(xt, cand)
    return out.reshape(FIELD_NUM, EMBED_DIM, ROWS).transpose(2, 0, 1)
